# baseline (speedup 1.0000x reference)
import functools

import jax
import jax.numpy as jnp
from jax import lax
from jax.experimental import pallas as pl
from jax.experimental.pallas import tpu as pltpu
from jax.experimental.pallas import tpu_sc as plsc

E = 8
TOP_K = 2
TM = 256
NEG_INF = -1e30

SC_CORES = 2
SC_SUBCORES = 16
NW = SC_CORES * SC_SUBCORES


def _router_body(x_ref, gw_ref, idx_ref, prob_ref):
    scores = jax.lax.dot_general(
        x_ref[...], gw_ref[...], (((1,), (1,)), ((), ())),
        preferred_element_type=jnp.float32,
        precision=jax.lax.Precision.DEFAULT)
    tr = scores.shape[0]
    iota = jax.lax.broadcasted_iota(jnp.int32, (tr, E), 1)
    m1 = jnp.max(scores, axis=1, keepdims=True)
    a1 = jnp.min(jnp.where(scores == m1, iota, E), axis=1, keepdims=True)
    masked = jnp.where(iota == a1, NEG_INF, scores)
    m2 = jnp.max(masked, axis=1, keepdims=True)
    a2 = jnp.min(jnp.where(masked == m2, iota, E), axis=1, keepdims=True)
    p1 = 1.0 / (1.0 + jnp.exp(m2 - m1))
    p2 = 1.0 - p1
    idx_ref[...] = jnp.concatenate([a1, a2], axis=1)
    prob_ref[...] = jnp.concatenate([p1, p2], axis=1)


def _router(x_flat, gate_w):
    n, d = x_flat.shape
    tr = 1024
    return pl.pallas_call(
        _router_body,
        grid=(n // tr,),
        in_specs=[
            pl.BlockSpec((tr, d), lambda i: (i, 0)),
            pl.BlockSpec((E, d), lambda i: (0, 0)),
        ],
        out_specs=[
            pl.BlockSpec((tr, TOP_K), lambda i: (i, 0)),
            pl.BlockSpec((tr, TOP_K), lambda i: (i, 0)),
        ],
        out_shape=[
            jax.ShapeDtypeStruct((n, TOP_K), jnp.int32),
            jax.ShapeDtypeStruct((n, TOP_K), jnp.float32),
        ],
    )(x_flat, gate_w)


def _build_dispatch(sel_idx, sel_prob, n, np_total):
    e_flat = sel_idx.reshape(-1)
    oh = (e_flat[:, None] == jnp.arange(E, dtype=jnp.int32)[None, :]
          ).astype(jnp.int32)
    cum = jnp.cumsum(oh, axis=0)
    counts = cum[-1]
    rank = ((cum - oh) * oh).sum(axis=1)
    padded = ((counts + TM - 1) // TM) * TM
    ends = jnp.cumsum(padded)
    starts = ends - padded
    pos_flat = (starts[e_flat] + rank).astype(jnp.int32)
    tok = (jnp.arange(n * TOP_K, dtype=jnp.int32) // TOP_K).astype(jnp.int32)
    src_token, w_sorted = _sc_dispatch_scatter(
        pos_flat, tok, sel_prob.reshape(-1), np_total)
    tile_starts = jnp.arange(np_total // TM, dtype=jnp.int32) * TM
    tile_expert = jnp.clip(
        jnp.searchsorted(ends, tile_starts, side="right"), 0, E - 1
    ).astype(jnp.int32)
    return src_token, w_sorted, tile_expert, pos_flat


def _sc_dispatch_scatter(pos_flat, tok, probs, np_total):
    na = pos_flat.shape[0]
    nsub = SC_SUBCORES
    a_per_w = na // nsub
    z_per_w = np_total // nsub
    lanes = 16
    mesh = plsc.VectorSubcoreMesh(core_axis_name="c", subcore_axis_name="s")

    @functools.partial(
        pl.kernel, mesh=mesh,
        out_type=[
            jax.ShapeDtypeStruct((np_total,), jnp.int32),
            jax.ShapeDtypeStruct((np_total,), jnp.float32),
        ],
        scratch_types=[
            pltpu.VMEM((a_per_w,), jnp.int32),
            pltpu.VMEM((a_per_w,), jnp.int32),
            pltpu.VMEM((a_per_w,), jnp.float32),
            pltpu.VMEM((z_per_w,), jnp.int32),
            pltpu.VMEM((z_per_w,), jnp.float32),
        ],
    )
    def k(pf_hbm, tok_hbm, pr_hbm, st_hbm, ws_hbm, pos_v, tok_v, pr_v,
          z_v, zf_v):
        cid = lax.axis_index("c")
        sid = lax.axis_index("s")

        @pl.when(cid == 0)
        def _():
            @pl.loop(0, z_per_w, step=lanes)
            def _(i):
                z_v.at[pl.ds(i, lanes)][...] = jnp.zeros((lanes,), jnp.int32)
                zf_v.at[pl.ds(i, lanes)][...] = jnp.zeros((lanes,), jnp.float32)

            zbase = sid * z_per_w
            pltpu.sync_copy(z_v, st_hbm.at[pl.ds(zbase, z_per_w)])
            pltpu.sync_copy(zf_v, ws_hbm.at[pl.ds(zbase, z_per_w)])

        plsc.subcore_barrier()

        @pl.when(cid == 0)
        def _():
            abase = sid * a_per_w
            pltpu.sync_copy(pf_hbm.at[pl.ds(abase, a_per_w)], pos_v)
            pltpu.sync_copy(tok_hbm.at[pl.ds(abase, a_per_w)], tok_v)
            pltpu.sync_copy(pr_hbm.at[pl.ds(abase, a_per_w)], pr_v)
            pltpu.sync_copy(tok_v, st_hbm.at[pos_v])
            pltpu.sync_copy(pr_v, ws_hbm.at[pos_v])

    return k(pos_flat, tok, probs)


def _sc_gather(x_flat, src_token, np_total):
    n, d = x_flat.shape
    rows_per_w = np_total // NW
    chunk = 96 if x_flat.dtype == jnp.bfloat16 else 48
    nchunks = rows_per_w // chunk
    assert nchunks * chunk == rows_per_w and nchunks % 2 == 0
    mesh = plsc.VectorSubcoreMesh(core_axis_name="c", subcore_axis_name="s")

    @functools.partial(
        pl.kernel, mesh=mesh,
        out_type=jax.ShapeDtypeStruct((np_total, d), x_flat.dtype),
        scratch_types=[
            pltpu.VMEM((rows_per_w,), jnp.int32),
            pltpu.VMEM((chunk, d), x_flat.dtype),
            pltpu.VMEM((chunk, d), x_flat.dtype),
            pltpu.SemaphoreType.DMA,
            pltpu.SemaphoreType.DMA,
        ],
    )
    def k(x_hbm, idx_hbm, out_hbm, idx_v, r0, r1, s0, s1):
        wid = lax.axis_index("s") * SC_CORES + lax.axis_index("c")
        base = wid * rows_per_w
        pltpu.sync_copy(idx_hbm.at[pl.ds(base, rows_per_w)], idx_v)

        def issue(buf, sem, c):
            pltpu.async_copy(x_hbm.at[idx_v.at[pl.ds(c * chunk, chunk)]],
                             buf, sem)

        def wait(buf, sem):
            pltpu.make_async_copy(
                x_hbm.at[idx_v.at[pl.ds(0, chunk)]], buf, sem).wait()

        issue(r0, s0, 0)

        @pl.loop(0, nchunks, step=2)
        def _(i):
            issue(r1, s1, i + 1)
            wait(r0, s0)
            pltpu.sync_copy(r0, out_hbm.at[pl.ds(base + i * chunk, chunk)])

            @pl.when(i + 2 < nchunks)
            def _():
                issue(r0, s0, i + 2)

            wait(r1, s1)
            pltpu.sync_copy(
                r1, out_hbm.at[pl.ds(base + (i + 1) * chunk, chunk)])

    return k(x_flat, src_token)


def _gmm_body(te_ref, x_ref, w12_ref, w3_ref, ws_ref, o_ref):
    xf = x_ref[...].astype(jnp.float32)
    h = jax.lax.dot_general(
        xf, w12_ref[0], (((1,), (1,)), ((), ())),
        preferred_element_type=jnp.float32)
    hh = h.shape[1] // 2
    h1 = h[:, :hh]
    h2 = h[:, hh:]
    hidden = h1 * jax.nn.sigmoid(h1) * h2
    o = jax.lax.dot_general(
        hidden, w3_ref[0], (((1,), (1,)), ((), ())),
        preferred_element_type=jnp.float32)
    o_ref[...] = o * ws_ref[0, 0][:, None]


def _gmm(x_sorted, fc12_w, fc3_w, w_sorted, tile_expert, np_total):
    d = x_sorted.shape[1]
    h2 = fc12_w.shape[1]
    h = fc3_w.shape[2]
    t_tiles = np_total // TM
    grid_spec = pltpu.PrefetchScalarGridSpec(
        num_scalar_prefetch=1,
        grid=(t_tiles,),
        in_specs=[
            pl.BlockSpec((TM, d), lambda i, te: (i, 0)),
            pl.BlockSpec((1, h2, d), lambda i, te: (te[i], 0, 0)),
            pl.BlockSpec((1, d, h), lambda i, te: (te[i], 0, 0)),
            pl.BlockSpec((1, 1, TM), lambda i, te: (i, 0, 0)),
        ],
        out_specs=pl.BlockSpec((TM, d), lambda i, te: (i, 0)),
    )
    return pl.pallas_call(
        _gmm_body,
        grid_spec=grid_spec,
        out_shape=jax.ShapeDtypeStruct((np_total, d), jnp.float32),
    )(tile_expert, x_sorted, fc12_w, fc3_w,
      w_sorted.reshape(t_tiles, 1, TM))


def _sc_combine(o_sorted, pos_flat, n):
    d = o_sorted.shape[1]
    rows_per_w = n // NW
    ct = 16
    nchunks = rows_per_w // ct
    assert nchunks * ct == rows_per_w and nchunks % 2 == 0
    lanes = 16
    mesh = plsc.VectorSubcoreMesh(core_axis_name="c", subcore_axis_name="s")

    @functools.partial(
        pl.kernel, mesh=mesh,
        out_type=jax.ShapeDtypeStruct((n, d), jnp.float32),
        scratch_types=[
            pltpu.VMEM((2 * rows_per_w,), jnp.int32),
            pltpu.VMEM((2 * ct, d), jnp.float32),
            pltpu.VMEM((2 * ct, d), jnp.float32),
            pltpu.VMEM((ct, d), jnp.float32),
            pltpu.SemaphoreType.DMA,
            pltpu.SemaphoreType.DMA,
        ],
    )
    def k(o_hbm, pf_hbm, out_hbm, idx_v, g0, g1, ob, s0, s1):
        wid = lax.axis_index("s") * SC_CORES + lax.axis_index("c")
        base = wid * rows_per_w
        pltpu.sync_copy(pf_hbm.at[pl.ds(2 * base, 2 * rows_per_w)], idx_v)

        def issue(buf, sem, c):
            pltpu.async_copy(
                o_hbm.at[idx_v.at[pl.ds(c * 2 * ct, 2 * ct)]], buf, sem)

        def wait(buf, sem):
            pltpu.make_async_copy(
                o_hbm.at[idx_v.at[pl.ds(0, 2 * ct)]], buf, sem).wait()

        def add_store(buf, c):
            @pl.loop(0, ct)
            def _(r):
                @pl.loop(0, d, step=lanes)
                def _(col):
                    cs = pl.ds(col, lanes)
                    ob.at[pl.ds(r, 1), cs][...] = (
                        buf.at[pl.ds(2 * r, 1), cs][...]
                        + buf.at[pl.ds(2 * r + 1, 1), cs][...])

            pltpu.sync_copy(ob, out_hbm.at[pl.ds(base + c * ct, ct)])

        issue(g0, s0, 0)

        @pl.loop(0, nchunks, step=2)
        def _(i):
            issue(g1, s1, i + 1)
            wait(g0, s0)
            add_store(g0, i)

            @pl.when(i + 2 < nchunks)
            def _():
                issue(g0, s0, i + 2)

            wait(g1, s1)
            add_store(g1, i + 1)

    return k(o_sorted, pos_flat)


def kernel(x, gate_w, fc12_w, fc3_w):
    b, t, d = x.shape
    n = b * t
    np_total = n * TOP_K + E * TM

    x_flat = x.reshape(n, d)
    sel_idx, sel_prob = _router(x_flat, gate_w)
    src_token, w_sorted, tile_expert, pos_flat = _build_dispatch(
        sel_idx, sel_prob, n, np_total)

    x_sorted = _sc_gather(x_flat, src_token, np_total)
    o_sorted = _gmm(x_sorted, fc12_w, fc3_w, w_sorted, tile_expert, np_total)

    out_flat = _sc_combine(o_sorted, pos_flat, n)
    return out_flat.reshape(b, t, d)


# speedup vs baseline: 1.3017x; 1.3017x over previous
import functools

import jax
import jax.numpy as jnp
from jax import lax
from jax.experimental import pallas as pl
from jax.experimental.pallas import tpu as pltpu
from jax.experimental.pallas import tpu_sc as plsc

E = 8
TOP_K = 2
TM = 256
NEG_INF = -1e30

SC_CORES = 2
SC_SUBCORES = 16
NW = SC_CORES * SC_SUBCORES
LANES = 16


def _router_body(x_ref, gw_ref, idx_ref, prob_ref):
    scores = jax.lax.dot_general(
        x_ref[...], gw_ref[...], (((1,), (1,)), ((), ())),
        preferred_element_type=jnp.float32,
        precision=jax.lax.Precision.DEFAULT)
    tr = scores.shape[0]
    iota = jax.lax.broadcasted_iota(jnp.int32, (tr, E), 1)
    m1 = jnp.max(scores, axis=1, keepdims=True)
    a1 = jnp.min(jnp.where(scores == m1, iota, E), axis=1, keepdims=True)
    masked = jnp.where(iota == a1, NEG_INF, scores)
    m2 = jnp.max(masked, axis=1, keepdims=True)
    a2 = jnp.min(jnp.where(masked == m2, iota, E), axis=1, keepdims=True)
    p1 = 1.0 / (1.0 + jnp.exp(m2 - m1))
    p2 = 1.0 - p1
    idx_ref[...] = jnp.concatenate([a1, a2], axis=1)
    prob_ref[...] = jnp.concatenate([p1, p2], axis=1)


def _router(x_flat, gate_w):
    n, d = x_flat.shape
    tr = 1024
    return pl.pallas_call(
        _router_body,
        grid=(n // tr,),
        in_specs=[
            pl.BlockSpec((tr, d), lambda i: (i, 0)),
            pl.BlockSpec((E, d), lambda i: (0, 0)),
        ],
        out_specs=[
            pl.BlockSpec((tr, TOP_K), lambda i: (i, 0)),
            pl.BlockSpec((tr, TOP_K), lambda i: (i, 0)),
        ],
        out_shape=[
            jax.ShapeDtypeStruct((n, TOP_K), jnp.int32),
            jax.ShapeDtypeStruct((n, TOP_K), jnp.float32),
        ],
    )(x_flat, gate_w)


def _build_dispatch(sel_idx, n, np_total):
    e_flat = sel_idx.reshape(-1)
    oh = (e_flat[:, None] == jnp.arange(E, dtype=jnp.int32)[None, :]
          ).astype(jnp.int32)
    cum = jnp.cumsum(oh, axis=0)
    counts = cum[-1]
    rank = ((cum - oh) * oh).sum(axis=1)
    padded = ((counts + TM - 1) // TM) * TM
    ends = jnp.cumsum(padded)
    starts = ends - padded
    pos_flat = (starts[e_flat] + rank).astype(jnp.int32)
    tile_starts = jnp.arange(np_total // TM, dtype=jnp.int32) * TM
    tile_expert = jnp.clip(
        jnp.searchsorted(ends, tile_starts, side="right"), 0, E - 1
    ).astype(jnp.int32)
    pos2 = pos_flat.reshape(n, TOP_K)
    return pos_flat, pos2[:, 0], pos2[:, 1], tile_expert


def _sc_dispatch_x(x_flat, pos_e, pos_o, np_total):
    n, d = x_flat.shape
    rows_per_w = n // NW
    ct = 32
    nchunks = rows_per_w // ct
    assert nchunks * ct == rows_per_w and nchunks % 2 == 0
    mesh = plsc.VectorSubcoreMesh(core_axis_name="c", subcore_axis_name="s")

    @functools.partial(
        pl.kernel, mesh=mesh,
        out_type=jax.ShapeDtypeStruct((np_total, d), jnp.float32),
        scratch_types=[
            pltpu.VMEM((ct, d), jnp.float32),
            pltpu.VMEM((ct, d), jnp.float32),
            pltpu.VMEM((ct,), jnp.int32),
            pltpu.VMEM((ct,), jnp.int32),
            pltpu.VMEM((ct,), jnp.int32),
            pltpu.VMEM((ct,), jnp.int32),
            pltpu.SemaphoreType.DMA,
            pltpu.SemaphoreType.DMA,
        ],
    )
    def k(x_hbm, pe_hbm, po_hbm, out_hbm, r0, r1, ie0, io0, ie1, io1, s0, s1):
        wid = lax.axis_index("s") * SC_CORES + lax.axis_index("c")
        base = wid * rows_per_w

        def load(buf, sem, c):
            pltpu.async_copy(
                x_hbm.at[pl.ds(base + c * ct, ct)], buf, sem)

        def wait(buf, sem):
            pltpu.make_async_copy(
                x_hbm.at[pl.ds(base, ct)], buf, sem).wait()

        def scatter(buf, ie, io, c):
            pltpu.sync_copy(pe_hbm.at[pl.ds(base + c * ct, ct)], ie)
            pltpu.sync_copy(po_hbm.at[pl.ds(base + c * ct, ct)], io)
            pltpu.sync_copy(buf, out_hbm.at[ie])
            pltpu.sync_copy(buf, out_hbm.at[io])

        load(r0, s0, 0)

        @pl.loop(0, nchunks, step=2)
        def _(i):
            load(r1, s1, i + 1)
            wait(r0, s0)
            scatter(r0, ie0, io0, i)

            @pl.when(i + 2 < nchunks)
            def _():
                load(r0, s0, i + 2)

            wait(r1, s1)
            scatter(r1, ie1, io1, i + 1)

    return k(x_flat, pos_e, pos_o)


def _gmm_body(te_ref, x_ref, w12_ref, w3_ref, o_ref):
    h = jax.lax.dot_general(
        x_ref[...], w12_ref[0], (((1,), (1,)), ((), ())),
        preferred_element_type=jnp.float32)
    hh = h.shape[1] // 2
    h1 = h[:, :hh]
    h2 = h[:, hh:]
    hidden = h1 * jax.nn.sigmoid(h1) * h2
    o = jax.lax.dot_general(
        hidden, w3_ref[0], (((1,), (1,)), ((), ())),
        preferred_element_type=jnp.float32)
    o_ref[...] = o


def _gmm(x_sorted, fc12_w, fc3_w, tile_expert, np_total):
    d = x_sorted.shape[1]
    h2 = fc12_w.shape[1]
    h = fc3_w.shape[2]
    t_tiles = np_total // TM
    grid_spec = pltpu.PrefetchScalarGridSpec(
        num_scalar_prefetch=1,
        grid=(t_tiles,),
        in_specs=[
            pl.BlockSpec((TM, d), lambda i, te: (i, 0)),
            pl.BlockSpec((1, h2, d), lambda i, te: (te[i], 0, 0)),
            pl.BlockSpec((1, d, h), lambda i, te: (te[i], 0, 0)),
        ],
        out_specs=pl.BlockSpec((TM, d), lambda i, te: (i, 0)),
    )
    return pl.pallas_call(
        _gmm_body,
        grid_spec=grid_spec,
        out_shape=jax.ShapeDtypeStruct((np_total, d), jnp.float32),
    )(tile_expert, x_sorted, fc12_w, fc3_w)


def _sc_gather(table, indices, n_out):
    d = table.shape[1]
    rows_per_w = n_out // NW
    chunk = 32
    nchunks = rows_per_w // chunk
    assert nchunks * chunk == rows_per_w and nchunks % 2 == 0
    mesh = plsc.VectorSubcoreMesh(core_axis_name="c", subcore_axis_name="s")

    @functools.partial(
        pl.kernel, mesh=mesh,
        out_type=jax.ShapeDtypeStruct((n_out, d), table.dtype),
        scratch_types=[
            pltpu.VMEM((rows_per_w,), jnp.int32),
            pltpu.VMEM((chunk, d), table.dtype),
            pltpu.VMEM((chunk, d), table.dtype),
            pltpu.SemaphoreType.DMA,
            pltpu.SemaphoreType.DMA,
        ],
    )
    def k(t_hbm, idx_hbm, out_hbm, idx_v, r0, r1, s0, s1):
        wid = lax.axis_index("s") * SC_CORES + lax.axis_index("c")
        base = wid * rows_per_w
        pltpu.sync_copy(idx_hbm.at[pl.ds(base, rows_per_w)], idx_v)

        def issue(buf, sem, c):
            pltpu.async_copy(t_hbm.at[idx_v.at[pl.ds(c * chunk, chunk)]],
                             buf, sem)

        def wait(buf, sem):
            pltpu.make_async_copy(
                t_hbm.at[idx_v.at[pl.ds(0, chunk)]], buf, sem).wait()

        issue(r0, s0, 0)

        @pl.loop(0, nchunks, step=2)
        def _(i):
            issue(r1, s1, i + 1)
            wait(r0, s0)
            pltpu.sync_copy(r0, out_hbm.at[pl.ds(base + i * chunk, chunk)])

            @pl.when(i + 2 < nchunks)
            def _():
                issue(r0, s0, i + 2)

            wait(r1, s1)
            pltpu.sync_copy(
                r1, out_hbm.at[pl.ds(base + (i + 1) * chunk, chunk)])

    return k(table, indices)


def _final_body(g_ref, p_ref, o_ref):
    d = o_ref.shape[1]
    o_ref[...] = (g_ref[:, :d] * p_ref[:, 0:1]
                  + g_ref[:, d:] * p_ref[:, 1:2])


def _final_combine(gathered, sel_prob, n, d):
    g2 = gathered.reshape(n, 2 * d)
    tr = 512
    return pl.pallas_call(
        _final_body,
        grid=(n // tr,),
        in_specs=[
            pl.BlockSpec((tr, 2 * d), lambda i: (i, 0)),
            pl.BlockSpec((tr, TOP_K), lambda i: (i, 0)),
        ],
        out_specs=pl.BlockSpec((tr, d), lambda i: (i, 0)),
        out_shape=jax.ShapeDtypeStruct((n, d), jnp.float32),
    )(g2, sel_prob)


def kernel(x, gate_w, fc12_w, fc3_w):
    b, t, d = x.shape
    n = b * t
    np_total = n * TOP_K + E * TM

    x_flat = x.reshape(n, d)
    sel_idx, sel_prob = _router(x_flat, gate_w)
    pos_flat, pos_e, pos_o, tile_expert = _build_dispatch(sel_idx, n, np_total)

    x_sorted = _sc_dispatch_x(x_flat, pos_e, pos_o, np_total)
    o_sorted = _gmm(x_sorted, fc12_w, fc3_w, tile_expert, np_total)

    gathered = _sc_gather(o_sorted, pos_flat, n * TOP_K)
    out_flat = _final_combine(gathered, sel_prob, n, d)
    return out_flat.reshape(b, t, d)


# speedup vs baseline: 1.4968x; 1.1499x over previous
import functools

import jax
import jax.numpy as jnp
from jax import lax
from jax.experimental import pallas as pl
from jax.experimental.pallas import tpu as pltpu
from jax.experimental.pallas import tpu_sc as plsc

E = 8
TOP_K = 2
TM = 256
NEG_INF = -1e30

SC_CORES = 2
SC_SUBCORES = 16
NW = SC_CORES * SC_SUBCORES
LANES = 16


def _router_body(x_ref, gw_ref, idx_ref, prob_ref):
    scores = jax.lax.dot_general(
        x_ref[...], gw_ref[...], (((1,), (1,)), ((), ())),
        preferred_element_type=jnp.float32,
        precision=jax.lax.Precision.DEFAULT)
    tr = scores.shape[0]
    iota = jax.lax.broadcasted_iota(jnp.int32, (tr, E), 1)
    m1 = jnp.max(scores, axis=1, keepdims=True)
    a1 = jnp.min(jnp.where(scores == m1, iota, E), axis=1, keepdims=True)
    masked = jnp.where(iota == a1, NEG_INF, scores)
    m2 = jnp.max(masked, axis=1, keepdims=True)
    a2 = jnp.min(jnp.where(masked == m2, iota, E), axis=1, keepdims=True)
    p1 = 1.0 / (1.0 + jnp.exp(m2 - m1))
    p2 = 1.0 - p1
    idx_ref[...] = jnp.concatenate([a1, a2], axis=1)
    prob_ref[...] = jnp.concatenate([p1, p2], axis=1)


def _router(x_flat, gate_w):
    n, d = x_flat.shape
    tr = 1024
    return pl.pallas_call(
        _router_body,
        grid=(n // tr,),
        in_specs=[
            pl.BlockSpec((tr, d), lambda i: (i, 0)),
            pl.BlockSpec((E, d), lambda i: (0, 0)),
        ],
        out_specs=[
            pl.BlockSpec((tr, TOP_K), lambda i: (i, 0)),
            pl.BlockSpec((tr, TOP_K), lambda i: (i, 0)),
        ],
        out_shape=[
            jax.ShapeDtypeStruct((n, TOP_K), jnp.int32),
            jax.ShapeDtypeStruct((n, TOP_K), jnp.float32),
        ],
    )(x_flat, gate_w)


def _build_dispatch(sel_idx, n, np_total):
    e_flat = sel_idx.reshape(-1)
    oh = (e_flat[:, None] == jnp.arange(E, dtype=jnp.int32)[None, :]
          ).astype(jnp.int32)
    cum = jnp.cumsum(oh, axis=0)
    counts = cum[-1]
    rank = ((cum - oh) * oh).sum(axis=1)
    padded = ((counts + TM - 1) // TM) * TM
    ends = jnp.cumsum(padded)
    starts = ends - padded
    pos_flat = (starts[e_flat] + rank).astype(jnp.int32)
    tile_starts = jnp.arange(np_total // TM, dtype=jnp.int32) * TM
    tile_expert = jnp.clip(
        jnp.searchsorted(ends, tile_starts, side="right"), 0, E - 1
    ).astype(jnp.int32)
    pos2 = pos_flat.reshape(n, TOP_K)
    return pos_flat, pos2[:, 0], pos2[:, 1], tile_expert


def _sc_dispatch_x(x_flat, pos_e, pos_o, np_total):
    n, d = x_flat.shape
    rows_per_w = n // NW
    ct = 32
    nchunks = rows_per_w // ct
    assert nchunks * ct == rows_per_w and nchunks % 2 == 0
    mesh = plsc.VectorSubcoreMesh(core_axis_name="c", subcore_axis_name="s")

    @functools.partial(
        pl.kernel, mesh=mesh,
        out_type=jax.ShapeDtypeStruct((np_total, d), jnp.float32),
        scratch_types=[
            pltpu.VMEM((ct, d), jnp.float32),
            pltpu.VMEM((ct, d), jnp.float32),
            pltpu.VMEM((ct,), jnp.int32),
            pltpu.VMEM((ct,), jnp.int32),
            pltpu.VMEM((ct,), jnp.int32),
            pltpu.VMEM((ct,), jnp.int32),
            pltpu.SemaphoreType.DMA,
            pltpu.SemaphoreType.DMA,
        ],
    )
    def k(x_hbm, pe_hbm, po_hbm, out_hbm, r0, r1, ie0, io0, ie1, io1, s0, s1):
        wid = lax.axis_index("s") * SC_CORES + lax.axis_index("c")
        base = wid * rows_per_w

        def load(buf, sem, c):
            pltpu.async_copy(
                x_hbm.at[pl.ds(base + c * ct, ct)], buf, sem)

        def wait(buf, sem):
            pltpu.make_async_copy(
                x_hbm.at[pl.ds(base, ct)], buf, sem).wait()

        def scatter(buf, ie, io, c):
            pltpu.sync_copy(pe_hbm.at[pl.ds(base + c * ct, ct)], ie)
            pltpu.sync_copy(po_hbm.at[pl.ds(base + c * ct, ct)], io)
            pltpu.sync_copy(buf, out_hbm.at[ie])
            pltpu.sync_copy(buf, out_hbm.at[io])

        load(r0, s0, 0)

        @pl.loop(0, nchunks, step=2)
        def _(i):
            load(r1, s1, i + 1)
            wait(r0, s0)
            scatter(r0, ie0, io0, i)

            @pl.when(i + 2 < nchunks)
            def _():
                load(r0, s0, i + 2)

            wait(r1, s1)
            scatter(r1, ie1, io1, i + 1)

    return k(x_flat, pos_e, pos_o)


def _gmm_body(te_ref, x_ref, w12_ref, w3_ref, o_ref):
    h = jax.lax.dot_general(
        x_ref[...], w12_ref[0], (((1,), (1,)), ((), ())),
        preferred_element_type=jnp.float32)
    hh = h.shape[1] // 2
    h1 = h[:, :hh]
    h2 = h[:, hh:]
    hidden = h1 * jax.nn.sigmoid(h1) * h2
    o = jax.lax.dot_general(
        hidden, w3_ref[0], (((1,), (1,)), ((), ())),
        preferred_element_type=jnp.float32)
    o_ref[...] = o


def _gmm(x_sorted, fc12_w, fc3_w, tile_expert, np_total):
    d = x_sorted.shape[1]
    h2 = fc12_w.shape[1]
    h = fc3_w.shape[2]
    t_tiles = np_total // TM
    grid_spec = pltpu.PrefetchScalarGridSpec(
        num_scalar_prefetch=1,
        grid=(t_tiles,),
        in_specs=[
            pl.BlockSpec((TM, d), lambda i, te: (i, 0)),
            pl.BlockSpec((1, h2, d), lambda i, te: (te[i], 0, 0)),
            pl.BlockSpec((1, d, h), lambda i, te: (te[i], 0, 0)),
        ],
        out_specs=pl.BlockSpec((TM, d), lambda i, te: (i, 0)),
    )
    return pl.pallas_call(
        _gmm_body,
        grid_spec=grid_spec,
        out_shape=jax.ShapeDtypeStruct((np_total, d), jnp.float32),
    )(tile_expert, x_sorted, fc12_w, fc3_w)


def _sc_gather(table, indices, n_out):
    d = table.shape[1]
    rows_per_w = n_out // NW
    chunk = 32
    nchunks = rows_per_w // chunk
    assert nchunks * chunk == rows_per_w and nchunks % 2 == 0
    mesh = plsc.VectorSubcoreMesh(core_axis_name="c", subcore_axis_name="s")

    @functools.partial(
        pl.kernel, mesh=mesh,
        out_type=jax.ShapeDtypeStruct((n_out, d), table.dtype),
        scratch_types=[
            pltpu.VMEM((rows_per_w,), jnp.int32),
            pltpu.VMEM((chunk, d), table.dtype),
            pltpu.VMEM((chunk, d), table.dtype),
            pltpu.SemaphoreType.DMA,
            pltpu.SemaphoreType.DMA,
        ],
    )
    def k(t_hbm, idx_hbm, out_hbm, idx_v, r0, r1, s0, s1):
        wid = lax.axis_index("s") * SC_CORES + lax.axis_index("c")
        base = wid * rows_per_w
        pltpu.sync_copy(idx_hbm.at[pl.ds(base, rows_per_w)], idx_v)

        def issue(buf, sem, c):
            pltpu.async_copy(t_hbm.at[idx_v.at[pl.ds(c * chunk, chunk)]],
                             buf, sem)

        def wait(buf, sem):
            pltpu.make_async_copy(
                t_hbm.at[idx_v.at[pl.ds(0, chunk)]], buf, sem).wait()

        issue(r0, s0, 0)

        @pl.loop(0, nchunks, step=2)
        def _(i):
            issue(r1, s1, i + 1)
            wait(r0, s0)
            pltpu.sync_copy(r0, out_hbm.at[pl.ds(base + i * chunk, chunk)])

            @pl.when(i + 2 < nchunks)
            def _():
                issue(r0, s0, i + 2)

            wait(r1, s1)
            pltpu.sync_copy(
                r1, out_hbm.at[pl.ds(base + (i + 1) * chunk, chunk)])

    return k(table, indices)


def _final_body(g0_ref, g1_ref, p_ref, o_ref):
    o_ref[...] = (g0_ref[...] * p_ref[:, 0:1]
                  + g1_ref[...] * p_ref[:, 1:2])


def _final_combine(gathered, sel_prob, n, d):
    tr = 1024
    return pl.pallas_call(
        _final_body,
        grid=(n // tr,),
        in_specs=[
            pl.BlockSpec((tr, d), lambda i: (i, 0)),
            pl.BlockSpec((tr, d), lambda i: (i + n // tr, 0)),
            pl.BlockSpec((tr, TOP_K), lambda i: (i, 0)),
        ],
        out_specs=pl.BlockSpec((tr, d), lambda i: (i, 0)),
        out_shape=jax.ShapeDtypeStruct((n, d), jnp.float32),
    )(gathered, gathered, sel_prob)


def kernel(x, gate_w, fc12_w, fc3_w):
    b, t, d = x.shape
    n = b * t
    np_total = n * TOP_K + E * TM

    x_flat = x.reshape(n, d)
    sel_idx, sel_prob = _router(x_flat, gate_w)
    pos_flat, pos_e, pos_o, tile_expert = _build_dispatch(sel_idx, n, np_total)

    x_sorted = _sc_dispatch_x(x_flat, pos_e, pos_o, np_total)
    o_sorted = _gmm(x_sorted, fc12_w, fc3_w, tile_expert, np_total)

    gathered = _sc_gather(
        o_sorted, jnp.concatenate([pos_e, pos_o]), n * TOP_K)
    out_flat = _final_combine(gathered, sel_prob, n, d)
    return out_flat.reshape(b, t, d)
